# baseline (speedup 1.0000x reference)
import functools

import jax
import jax.numpy as jnp
from jax import lax
from jax.experimental import pallas as pl
from jax.experimental.pallas import tpu as pltpu
from jax.experimental.pallas import tpu_sc as plsc


def _tc_repack(table_t, BLK=1024):
    C, V = table_t.shape

    def body(x_ref, o_ref):
        rr = lax.broadcasted_iota(jnp.int32, (C, C), 0)
        cc = lax.broadcasted_iota(jnp.int32, (C, C), 1)
        eye = (rr == cc).astype(jnp.float32)
        x = x_ref[...]
        t = lax.dot_general(
            x,
            eye,
            (((0,), (0,)), ((), ())),
            precision=lax.Precision.HIGHEST,
            preferred_element_type=jnp.float32,
        )
        y = t.reshape(BLK // 2, 2, C)
        o_ref[...] = jnp.concatenate([y[:, 0, :], y[:, 1, :]], axis=-1)

    grid = (V + BLK - 1) // BLK
    return pl.pallas_call(
        body,
        grid=(grid,),
        in_specs=[pl.BlockSpec((C, BLK), lambda i: (0, i))],
        out_specs=pl.BlockSpec((BLK // 2, 2 * C), lambda i: (i, 0)),
        out_shape=jax.ShapeDtypeStruct((V // 2, 2 * C), jnp.float32),
        compiler_params=pltpu.CompilerParams(
            dimension_semantics=("arbitrary",)
        ),
    )(table_t)


def _make_sc_gather(B, G, D2):
    info = plsc.get_sparse_core_info()
    NC, NS, L = info.num_cores, info.num_subcores, info.num_lanes
    NW = NC * NS
    assert B % (8 * NW) == 0
    b_per_w = B // NW
    mesh = plsc.VectorSubcoreMesh(core_axis_name="c", subcore_axis_name="s")
    D = D2 // 2

    @functools.partial(
        pl.kernel,
        mesh=mesh,
        out_type=jax.ShapeDtypeStruct((B, D2), jnp.float32),
        scratch_types=[
            pltpu.VMEM((b_per_w,), jnp.int32),
            pltpu.VMEM((b_per_w,), jnp.int32),
            pltpu.VMEM((b_per_w, D2), jnp.float32),
            pltpu.SemaphoreType.DMA,
        ],
        compiler_params=pltpu.CompilerParams(needs_layout_passes=False),
    )
    def gather(ids_hbm, lin_hbm, out_hbm, idx_v, pair_v, rows_v, sem):
        wid = lax.axis_index("s") * NC + lax.axis_index("c")
        base = wid * b_per_w
        pltpu.sync_copy(ids_hbm.at[pl.ds(base, b_per_w)], idx_v)

        def compute_pairs(i, carry):
            v = idx_v[pl.ds(i * L, L)]
            pair_v[pl.ds(i * L, L)] = lax.shift_right_logical(v, 1)
            return carry

        lax.fori_loop(0, b_per_w // L, compute_pairs, 0)
        pltpu.async_copy(lin_hbm.at[pair_v], rows_v, sem).wait()

        def extract(g, carry):
            v = idx_v[pl.ds(g * L, L)]
            for l in range(L):
                s = v[l]

                @pl.when((s & 1) == 1)
                def _():
                    i = g * L + l
                    for q in range(D // L):
                        rows_v[i, pl.ds(q * L, L)] = rows_v[
                            i, pl.ds(D + q * L, L)
                        ]

            return carry

        lax.fori_loop(0, b_per_w // L, extract, 0)
        pltpu.sync_copy(rows_v, out_hbm.at[pl.ds(base, b_per_w)])

    return gather


def kernel(item_ids, emb_weight):
    B, = item_ids.shape
    V, D = emb_weight.shape
    ids = item_ids.astype(jnp.int32)
    lin = _tc_repack(emb_weight.T)
    wide = _make_sc_gather(B, V // 2, 2 * D)(ids, lin)
    return wide[:, :D]


# speedup vs baseline: 1.6101x; 1.6101x over previous
import functools

import jax
import jax.numpy as jnp
from jax import lax
from jax.experimental import pallas as pl
from jax.experimental.pallas import tpu as pltpu
from jax.experimental.pallas import tpu_sc as plsc


def _tc_repack(table_t, BLK=1024):
    C, V = table_t.shape

    def body(x_ref, o_ref):
        rr = lax.broadcasted_iota(jnp.int32, (C, C), 0)
        cc = lax.broadcasted_iota(jnp.int32, (C, C), 1)
        eye = (rr == cc).astype(jnp.float32)
        x = x_ref[...]
        t = lax.dot_general(
            x,
            eye,
            (((0,), (0,)), ((), ())),
            precision=lax.Precision.HIGHEST,
            preferred_element_type=jnp.float32,
        )
        y = t.reshape(BLK // 2, 2, C)
        o_ref[...] = jnp.concatenate([y[:, 0, :], y[:, 1, :]], axis=-1)

    grid = (V + BLK - 1) // BLK
    return pl.pallas_call(
        body,
        grid=(grid,),
        in_specs=[pl.BlockSpec((C, BLK), lambda i: (0, i))],
        out_specs=pl.BlockSpec((BLK // 2, 2 * C), lambda i: (i, 0)),
        out_shape=jax.ShapeDtypeStruct((V // 2, 2 * C), jnp.float32),
        compiler_params=pltpu.CompilerParams(
            dimension_semantics=("arbitrary",)
        ),
    )(table_t)


def _make_sc_gather(B, G, D2):
    info = plsc.get_sparse_core_info()
    NC, NS, L = info.num_cores, info.num_subcores, info.num_lanes
    NW = NC * NS
    assert B % (8 * NW) == 0
    b_per_w = B // NW
    mesh = plsc.VectorSubcoreMesh(core_axis_name="c", subcore_axis_name="s")
    D = D2 // 2

    @functools.partial(
        pl.kernel,
        mesh=mesh,
        out_type=jax.ShapeDtypeStruct((B, D2), jnp.float32),
        scratch_types=[
            pltpu.VMEM((b_per_w,), jnp.int32),
            pltpu.VMEM((b_per_w,), jnp.int32),
            pltpu.VMEM((b_per_w, D2), jnp.float32),
            pltpu.SemaphoreType.DMA,
        ],
        compiler_params=pltpu.CompilerParams(needs_layout_passes=False),
    )
    def gather(ids_hbm, lin_hbm, out_hbm, idx_v, pair_v, rows_v, sem):
        wid = lax.axis_index("s") * NC + lax.axis_index("c")
        base = wid * b_per_w
        pltpu.sync_copy(ids_hbm.at[pl.ds(base, b_per_w)], idx_v)

        def compute_pairs(i, carry):
            v = idx_v[pl.ds(i * L, L)]
            pair_v[pl.ds(i * L, L)] = lax.shift_right_logical(v, 1)
            return carry

        lax.fori_loop(0, b_per_w // L, compute_pairs, 0)
        pltpu.async_copy(lin_hbm.at[pair_v], rows_v, sem).wait()

        def extract(g, carry):
            v = idx_v[pl.ds(g * L, L)]
            for l in range(L):
                s = v[l]

                @pl.when((s & 1) == 1)
                def _():
                    i = g * L + l
                    for q in range(D // L):
                        rows_v[i, pl.ds(q * L, L)] = rows_v[
                            i, pl.ds(D + q * L, L)
                        ]

            return carry

        lax.fori_loop(0, b_per_w // L, extract, 0)
        pltpu.sync_copy(rows_v, out_hbm.at[pl.ds(base, b_per_w)])

    return gather


def kernel(item_ids, emb_weight):
    B, = item_ids.shape
    V, D = emb_weight.shape
    ids = item_ids.astype(jnp.int32)
    lin = emb_weight.reshape(V // 2, 2 * D)
    wide = _make_sc_gather(B, V // 2, 2 * D)(ids, lin)
    return wide[:, :D]


# speedup vs baseline: 1.9811x; 1.2304x over previous
import functools

import jax
import jax.numpy as jnp
from jax import lax
from jax.experimental import pallas as pl
from jax.experimental.pallas import tpu as pltpu
from jax.experimental.pallas import tpu_sc as plsc

NBUF = 2
BCAP = 16
SCAP = 224
NBK = 256


def _make_sc_gather(B, C, V):
    info = plsc.get_sparse_core_info()
    NC, NS, L = info.num_cores, info.num_subcores, info.num_lanes
    NW = NC * NS
    NT = (V + 127) // 128
    TPW = (NT + NW - 1) // NW
    OUTER = (TPW + NBUF - 1) // NBUF
    mesh = plsc.VectorSubcoreMesh(core_axis_name="c", subcore_axis_name="s")
    thresh = SCAP - BCAP - 2 * L

    @functools.partial(
        pl.kernel,
        mesh=mesh,
        out_type=jax.ShapeDtypeStruct((B + 8, 2 * C), jnp.float32),
        scratch_types=[
            pltpu.VMEM((B,), jnp.int32),
            pltpu.VMEM((NBK, BCAP), jnp.int32),
            pltpu.VMEM((NBK, BCAP), jnp.int32),
            pltpu.VMEM((NBK,), jnp.int32),
            *[pltpu.VMEM((C, 2 * C), jnp.float32) for _ in range(NBUF)],
            pltpu.VMEM((SCAP, 2 * C), jnp.float32),
            pltpu.VMEM((SCAP,), jnp.int32),
            pltpu.VMEM((2 * L,), jnp.int32),
            pltpu.VMEM((2 * L,), jnp.int32),
            *[pltpu.SemaphoreType.DMA for _ in range(NBUF)],
            pltpu.SemaphoreType.DMA,
        ],
        compiler_params=pltpu.CompilerParams(needs_layout_passes=False),
    )
    def gather(ids_hbm, t_hbm, out_hbm, ids_v, bids, bpos, bcnt, *rest):
        chunks = rest[:NBUF]
        rows_v, spos_v, tmp_i, tmp_p = rest[NBUF : NBUF + 4]
        dsems = rest[NBUF + 4 : 2 * NBUF + 4]
        ssem = rest[2 * NBUF + 4]
        w = lax.axis_index("s") * NC + lax.axis_index("c")
        lane = lax.iota(jnp.int32, L)
        lane0 = lane == 0
        zeros = jnp.zeros((L,), jnp.int32)

        pltpu.sync_copy(ids_hbm, ids_v)

        def init_cnt(i, c):
            bcnt[pl.ds(i * L, L)] = zeros
            return c

        lax.fori_loop(0, NBK // L, init_cnt, 0)

        def init_spos(i, c):
            spos_v[pl.ds(i * L, L)] = zeros + B
            return c

        lax.fori_loop(0, SCAP // L, init_spos, 0)

        def scan_vreg(i, c):
            v = ids_v[pl.ds(i * L, L)]
            t = lax.shift_right_logical(v, 7)
            mine = (t & (NW - 1)) == w
            n = plsc.all_reduce_population_count(mine)[0]

            @pl.when(n > 0)
            def _():
                plsc.store_compressed(tmp_i.at[pl.ds(0, L)], v, mask=mine)
                plsc.store_compressed(tmp_p.at[pl.ds(0, L)], lane + i * L, mask=mine)

            def put(k, c2):
                sid = tmp_i[pl.ds(k, L)][0]
                sp = tmp_p[pl.ds(k, L)][0]
                bk = lax.shift_right_logical(sid, 12)
                cb = plsc.load_gather(bcnt, [zeros + bk])[0]

                @pl.when(cb < BCAP)
                def _():
                    plsc.store_scatter(
                        bids, [zeros + bk, zeros + cb], zeros + sid, mask=lane0
                    )
                    plsc.store_scatter(
                        bpos, [zeros + bk, zeros + cb], zeros + sp, mask=lane0
                    )

                plsc.store_scatter(bcnt, [zeros + bk], zeros + cb + 1, mask=lane0)
                return c2

            lax.fori_loop(0, n, put, 0)
            return c

        lax.fori_loop(0, B // L, scan_vreg, 0)

        def tile_start(tnum):
            return pl.multiple_of(jnp.minimum(tnum, NT - 1) * 128, 128)

        def issue(ci, b):
            tnum = ci * NW + w
            start = tile_start(tnum)
            pltpu.async_copy(
                t_hbm.at[:, pl.ds(start, 128)], chunks[b], dsems[b]
            )

        for b in range(NBUF):
            issue(jnp.int32(b), b)

        def extract_item(sid, sp, start, s, b):
            wl = sid - start
            for q in range(C // L):
                x = plsc.load_gather(chunks[b], [lane + q * L, zeros + wl])
                rows_v[s, pl.ds(q * L, L)] = x
            plsc.store_scatter(spos_v, [zeros + s], zeros + sp, mask=lane0)
            return s + 1

        def flush(s, always=False):
            do = (s > 0) if always else (s >= thresh)

            @pl.when(do)
            def _():
                pltpu.async_copy(rows_v, out_hbm.at[spos_v], ssem).wait()

                def reset(i, c):
                    spos_v[pl.ds(i * L, L)] = zeros + B
                    return c

                lax.fori_loop(0, SCAP // L, reset, 0)

            return jnp.where(do, 0, s)

        def slow_path(ci, tnum, start, b, s, active):
            nv = jnp.where(active, B // L, 0)

            def svreg(i, s2):
                v = ids_v[pl.ds(i * L, L)]
                m = lax.shift_right_logical(v, 7) == tnum
                n = plsc.all_reduce_population_count(m)[0]

                @pl.when(n > 0)
                def _():
                    plsc.store_compressed(tmp_i.at[pl.ds(0, L)], v, mask=m)
                    plsc.store_compressed(
                        tmp_p.at[pl.ds(0, L)], lane + i * L, mask=m
                    )

                def put(k, s3):
                    sid = tmp_i[pl.ds(k, L)][0]
                    sp = tmp_p[pl.ds(k, L)][0]
                    return extract_item(sid, sp, start, s3, b)

                s2 = lax.fori_loop(0, n, put, s2)
                return flush(s2)

            return lax.fori_loop(0, nv, svreg, s)

        def outer_body(co, s):
            for b in range(NBUF):
                ci = co * NBUF + b
                live = ci < TPW

                @pl.when(live)
                def _():
                    pltpu.make_async_copy(
                        t_hbm.at[:, pl.ds(0, 128)], chunks[b], dsems[b]
                    ).wait()

                tnum = ci * NW + w
                start = tile_start(tnum)
                nc = plsc.load_gather(bcnt, [zeros + ci])[0]
                nfast = jnp.minimum(
                    jnp.where(live, nc, 0), jnp.int32(BCAP)
                )

                def fast(k, s2):
                    sid = plsc.load_gather(bids, [zeros + ci, zeros + k])[0]
                    sp = plsc.load_gather(bpos, [zeros + ci, zeros + k])[0]
                    return extract_item(sid, sp, start, s2, b)

                s = lax.fori_loop(0, nfast, fast, s)
                s = flush(s)
                s = slow_path(ci, tnum, start, b, s, live & (nc > BCAP))

                @pl.when(ci + NBUF < TPW)
                def _():
                    issue(ci + NBUF, b)

            return s

        s = lax.fori_loop(0, OUTER, outer_body, jnp.int32(0))
        flush(s, always=True)

    return gather


def kernel(item_ids, emb_weight):
    B, = item_ids.shape
    V, D = emb_weight.shape
    ids = item_ids.astype(jnp.int32)
    wide = _make_sc_gather(B, D, V)(ids, emb_weight.T)
    return wide[:B, :D]
